# baseline (speedup 1.0000x reference)
import functools

import numpy as np
import jax
import jax.numpy as jnp
from jax import lax
from jax.experimental import pallas as pl
from jax.experimental.pallas import tpu as pltpu
from jax.experimental.pallas import tpu_sc as plsc

_N1 = 256
_LS = 65536
_W = 1024
_NC = 2
_NS = 16
_NW = _NC * _NS
_CW = 128

_PREC = lax.Precision.DEFAULT


@functools.lru_cache(maxsize=None)
def _dft_consts():
    k = np.arange(_N1, dtype=np.float64)
    ang = -2.0 * np.pi / _N1 * np.outer(k, k)
    dr = np.cos(ang).astype(np.float32)
    di = np.sin(ang).astype(np.float32)
    angt = -2.0 * np.pi / _LS * np.outer(k, k)
    tr = np.cos(angt).astype(np.float32)
    ti = np.sin(angt).astype(np.float32)
    t = np.arange(_W, dtype=np.float64)
    angm = 2.0 * np.pi / _W * np.outer(t, t)
    mr = (np.cos(angm) / _W).astype(np.float32)
    mi = (np.sin(angm) / _W).astype(np.float32)
    btop = np.empty((_W, 2 * _W), np.float32)
    bbot = np.empty((_W, 2 * _W), np.float32)
    btop[:, 0::2] = mr
    btop[:, 1::2] = mi
    bbot[:, 0::2] = -mi
    bbot[:, 1::2] = mr
    return dr, di, tr, ti, btop, bbot


def _fft_body(x_ref, dr_ref, di_ref, tr_ref, ti_ref, er_ref, ei_ref):
    a2 = x_ref[0]
    dr = dr_ref[...]
    di = di_ref[...]
    gr = jnp.dot(dr, a2, precision=_PREC, preferred_element_type=jnp.float32)
    gi = jnp.dot(di, a2, precision=_PREC, preferred_element_type=jnp.float32)
    tr = tr_ref[...]
    ti = ti_ref[...]
    hr = gr * tr - gi * ti
    hi = gr * ti + gi * tr
    er_ref[0] = (jnp.dot(hr, dr, precision=_PREC, preferred_element_type=jnp.float32)
                 - jnp.dot(hi, di, precision=_PREC, preferred_element_type=jnp.float32))
    ei_ref[0] = (jnp.dot(hr, di, precision=_PREC, preferred_element_type=jnp.float32)
                 + jnp.dot(hi, dr, precision=_PREC, preferred_element_type=jnp.float32))


def _idft_body(zre_ref, zim_ref, g_ref, btop_ref, bbot_ref, o_ref):
    gg = g_ref[...]
    pr = (zre_ref[0] * gg).astype(jnp.bfloat16)
    pi = (zim_ref[0] * gg).astype(jnp.bfloat16)
    acc = jnp.dot(pr, btop_ref[...], preferred_element_type=jnp.float32)
    o_ref[0] = acc + jnp.dot(pi, bbot_ref[...], preferred_element_type=jnp.float32)


def _make_sc_gather(f, width):
    bands_per_w = f // _NW
    half = _W // 2
    mesh = plsc.VectorSubcoreMesh(core_axis_name="c", subcore_axis_name="s",
                                  num_cores=_NC, num_subcores=_NS)

    @functools.partial(
        pl.kernel,
        out_type=jax.ShapeDtypeStruct((f, _W, width), jnp.float32),
        mesh=mesh,
        compiler_params=pltpu.CompilerParams(use_tc_tiling_on_sc=False,
                                             needs_layout_passes=False),
        scratch_types=[
            [pltpu.VMEM((16,), jnp.int32) for _ in range(3)],
            [pltpu.VMEM((_W, width), jnp.float32) for _ in range(3)],
            pltpu.SemaphoreType.DMA,
            pltpu.SemaphoreType.DMA,
        ],
    )
    def sc_gather(table_hbm, ix_hbm, out_hbm, tp_vs, win_vs, sem_in, sem_out):
        wid = lax.axis_index("s") * _NC + lax.axis_index("c")
        lane = lax.iota(jnp.int32, 16)
        for j in range(bands_per_w):
            pltpu.sync_copy(ix_hbm.at[wid * bands_per_w + j, pl.ds(0, 16)],
                            tp_vs[j])
        copies = []
        for j in range(bands_per_w):
            tp = jnp.max(jnp.where(lane == 0, tp_vs[j][...], 0))
            copies.append(
                pltpu.async_copy(table_hbm.at[pl.ds(tp, _W)], win_vs[j],
                                 sem_in))
        out_copies = []
        for j in range(bands_per_w):
            band = wid * bands_per_w + j
            copies[j].wait()
            out_copies.append(pltpu.async_copy(
                win_vs[j].at[pl.ds(half, half)],
                out_hbm.at[band, pl.ds(0, half)], sem_out))
            out_copies.append(pltpu.async_copy(
                win_vs[j].at[pl.ds(0, half)],
                out_hbm.at[band, pl.ds(half, half)], sem_out))
        for cp in out_copies:
            cp.wait()

    return sc_gather


def kernel(x, g, win_ix):
    b, c, ls = x.shape
    f, w = g.shape
    bc = b * c
    assert ls == _LS and w == _W

    dr, di, tr, ti, btop, bbot = _dft_consts()

    x3 = x.reshape(bc, _N1, _N1)
    full = pl.BlockSpec((_N1, _N1), lambda i: (0, 0))
    ere, eim = pl.pallas_call(
        _fft_body,
        grid=(bc,),
        in_specs=[pl.BlockSpec((1, _N1, _N1), lambda i: (i, 0, 0)),
                  full, full, full, full],
        out_specs=[pl.BlockSpec((1, _N1, _N1), lambda i: (i, 0, 0))] * 2,
        out_shape=[jax.ShapeDtypeStruct((bc, _N1, _N1), jnp.float32)] * 2,
    )(x3, dr, di, tr, ti)

    table = jnp.stack([ere, eim], axis=-1).transpose(2, 1, 0, 3).reshape(ls, bc * 2)
    tablep = jnp.concatenate(
        [table[ls - _W // 2:], table, table[:_W // 2]], axis=0)

    ix = win_ix.astype(jnp.int32)
    z = _make_sc_gather(f, bc * 2)(tablep, ix)
    gt = z.reshape(f, w, bc, 2).transpose(2, 3, 0, 1)
    zre, zim = gt[:, 0], gt[:, 1]

    fullg = pl.BlockSpec((f, w), lambda i: (0, 0))
    fullm = pl.BlockSpec((w, 2 * w), lambda i: (0, 0))
    out = pl.pallas_call(
        _idft_body,
        grid=(bc,),
        in_specs=[pl.BlockSpec((1, f, w), lambda i: (i, 0, 0)),
                  pl.BlockSpec((1, f, w), lambda i: (i, 0, 0)),
                  fullg, fullm, fullm],
        out_specs=pl.BlockSpec((1, f, 2 * w), lambda i: (i, 0, 0)),
        out_shape=jax.ShapeDtypeStruct((bc, f, 2 * w), jnp.float32),
    )(zre, zim, g,
      jnp.asarray(btop).astype(jnp.bfloat16),
      jnp.asarray(bbot).astype(jnp.bfloat16))

    return out.reshape(b, c, f, w, 2)


# speedup vs baseline: 1.2340x; 1.2340x over previous
import functools

import numpy as np
import jax
import jax.numpy as jnp
from jax import lax
from jax.experimental import pallas as pl
from jax.experimental.pallas import tpu as pltpu
from jax.experimental.pallas import tpu_sc as plsc

_N1 = 256
_LS = 65536
_W = 1024
_NC = 2
_NS = 16
_NW = _NC * _NS
_CW = 128

_PREC = lax.Precision.DEFAULT


@functools.lru_cache(maxsize=None)
def _dft_consts():
    k = np.arange(_N1, dtype=np.float64)
    ang = -2.0 * np.pi / _N1 * np.outer(k, k)
    dr = np.cos(ang).astype(np.float32)
    di = np.sin(ang).astype(np.float32)
    angt = -2.0 * np.pi / _LS * np.outer(k, k)
    tr = np.cos(angt).astype(np.float32)
    ti = np.sin(angt).astype(np.float32)
    t = np.arange(_W, dtype=np.float64)
    angm = 2.0 * np.pi / _W * np.outer(t, t)
    mr = (np.cos(angm) / _W).astype(np.float32)
    mi = (np.sin(angm) / _W).astype(np.float32)
    btop = np.empty((_W, 2 * _W), np.float32)
    bbot = np.empty((_W, 2 * _W), np.float32)
    btop[:, 0::2] = mr
    btop[:, 1::2] = mi
    bbot[:, 0::2] = -mi
    bbot[:, 1::2] = mr
    return dr, di, tr, ti, btop, bbot


def _fft_body(x_ref, dr_ref, di_ref, tr_ref, ti_ref, er_ref, ei_ref):
    a2 = x_ref[0]
    dr = dr_ref[...]
    di = di_ref[...]
    gr = jnp.dot(dr, a2, precision=_PREC, preferred_element_type=jnp.float32)
    gi = jnp.dot(di, a2, precision=_PREC, preferred_element_type=jnp.float32)
    tr = tr_ref[...]
    ti = ti_ref[...]
    hr = gr * tr - gi * ti
    hi = gr * ti + gi * tr
    er_ref[0] = (jnp.dot(hr, dr, precision=_PREC, preferred_element_type=jnp.float32)
                 - jnp.dot(hi, di, precision=_PREC,
                           preferred_element_type=jnp.float32)).astype(jnp.bfloat16)
    ei_ref[0] = (jnp.dot(hr, di, precision=_PREC, preferred_element_type=jnp.float32)
                 + jnp.dot(hi, dr, precision=_PREC,
                           preferred_element_type=jnp.float32)).astype(jnp.bfloat16)


def _idft_body(zre_ref, zim_ref, g_ref, btop_ref, bbot_ref, o_ref):
    gg = g_ref[...].astype(jnp.bfloat16)
    nsig = zre_ref.shape[0]
    pr = (zre_ref[...] * gg[None]).reshape(nsig * gg.shape[0], gg.shape[1])
    pi = (zim_ref[...] * gg[None]).reshape(nsig * gg.shape[0], gg.shape[1])
    acc = jnp.dot(pr, btop_ref[...], preferred_element_type=jnp.float32)
    acc = acc + jnp.dot(pi, bbot_ref[...], preferred_element_type=jnp.float32)
    o_ref[...] = acc.reshape(o_ref.shape)


def _make_sc_gather(f, width):
    bands_per_w = f // _NW
    half = _W // 2
    mesh = plsc.VectorSubcoreMesh(core_axis_name="c", subcore_axis_name="s",
                                  num_cores=_NC, num_subcores=_NS)

    @functools.partial(
        pl.kernel,
        out_type=jax.ShapeDtypeStruct((f, _W, width), jnp.bfloat16),
        mesh=mesh,
        compiler_params=pltpu.CompilerParams(use_tc_tiling_on_sc=False,
                                             needs_layout_passes=False),
        scratch_types=[
            [pltpu.VMEM((16,), jnp.int32) for _ in range(3)],
            [pltpu.VMEM((_W, width), jnp.bfloat16) for _ in range(3)],
            pltpu.SemaphoreType.DMA,
            pltpu.SemaphoreType.DMA,
        ],
    )
    def sc_gather(table_hbm, ix_hbm, out_hbm, tp_vs, win_vs, sem_in, sem_out):
        wid = lax.axis_index("s") * _NC + lax.axis_index("c")
        lane = lax.iota(jnp.int32, 16)
        for j in range(bands_per_w):
            pltpu.sync_copy(ix_hbm.at[wid * bands_per_w + j], tp_vs[j])
        copies = []
        for j in range(bands_per_w):
            tp = jnp.max(jnp.where(lane == 0, tp_vs[j][...], 0))
            copies.append(
                pltpu.async_copy(table_hbm.at[pl.ds(tp, _W)], win_vs[j],
                                 sem_in))
        out_copies = []
        for j in range(bands_per_w):
            band = wid * bands_per_w + j
            copies[j].wait()
            out_copies.append(pltpu.async_copy(
                win_vs[j].at[pl.ds(half, half)],
                out_hbm.at[band, pl.ds(0, half)], sem_out))
            out_copies.append(pltpu.async_copy(
                win_vs[j].at[pl.ds(0, half)],
                out_hbm.at[band, pl.ds(half, half)], sem_out))
        for cp in out_copies:
            cp.wait()

    return sc_gather


def kernel(x, g, win_ix):
    b, c, ls = x.shape
    f, w = g.shape
    bc = b * c
    assert ls == _LS and w == _W

    dr, di, tr, ti, btop, bbot = _dft_consts()

    x3 = x.reshape(bc, _N1, _N1)
    full = pl.BlockSpec((_N1, _N1), lambda i: (0, 0))
    ere, eim = pl.pallas_call(
        _fft_body,
        grid=(bc,),
        in_specs=[pl.BlockSpec((1, _N1, _N1), lambda i: (i, 0, 0)),
                  full, full, full, full],
        out_specs=[pl.BlockSpec((1, _N1, _N1), lambda i: (i, 0, 0))] * 2,
        out_shape=[jax.ShapeDtypeStruct((bc, _N1, _N1), jnp.bfloat16)] * 2,
    )(x3, dr, di, tr, ti)

    k2hi = (ls // 2 + _W // 2) // _N1
    low = jnp.stack([ere[:, :, :k2hi], eim[:, :, :k2hi]], axis=-1)
    hal = jnp.stack([ere[:, :, _N1 - 2:], eim[:, :, _N1 - 2:]], axis=-1)
    table = low.transpose(2, 1, 0, 3).reshape(k2hi * _N1, bc * 2)
    halo = hal.transpose(2, 1, 0, 3).reshape(2 * _N1, bc * 2)
    tablep = jnp.concatenate([halo, table], axis=0)

    ix = win_ix[:, :16].astype(jnp.int32)
    z = _make_sc_gather(f, bc * 2)(tablep, ix)
    gt = z.reshape(f, w, bc, 2).transpose(2, 3, 0, 1)
    zre, zim = gt[:, 0], gt[:, 1]

    rows = 4
    fullg = pl.BlockSpec((f, w), lambda i: (0, 0))
    fullm = pl.BlockSpec((w, 2 * w), lambda i: (0, 0))
    out = pl.pallas_call(
        _idft_body,
        grid=(bc // rows,),
        in_specs=[pl.BlockSpec((rows, f, w), lambda i: (i, 0, 0)),
                  pl.BlockSpec((rows, f, w), lambda i: (i, 0, 0)),
                  fullg, fullm, fullm],
        out_specs=pl.BlockSpec((rows, f, 2 * w), lambda i: (i, 0, 0)),
        out_shape=jax.ShapeDtypeStruct((bc, f, 2 * w), jnp.float32),
    )(zre, zim, g,
      jnp.asarray(btop).astype(jnp.bfloat16),
      jnp.asarray(bbot).astype(jnp.bfloat16))

    return out.reshape(b, c, f, w, 2)


# speedup vs baseline: 1.2356x; 1.0013x over previous
import functools

import numpy as np
import jax
import jax.numpy as jnp
from jax import lax
from jax.experimental import pallas as pl
from jax.experimental.pallas import tpu as pltpu
from jax.experimental.pallas import tpu_sc as plsc

_N1 = 256
_LS = 65536
_W = 1024
_NC = 2
_NS = 16
_NW = _NC * _NS
_CW = 128

_PREC = lax.Precision.DEFAULT


@functools.lru_cache(maxsize=None)
def _dft_consts():
    k = np.arange(_N1, dtype=np.float64)
    ang = -2.0 * np.pi / _N1 * np.outer(k, k)
    dr = np.cos(ang).astype(np.float32)
    di = np.sin(ang).astype(np.float32)
    angt = -2.0 * np.pi / _LS * np.outer(k, k)
    tr = np.cos(angt).astype(np.float32)
    ti = np.sin(angt).astype(np.float32)
    t = np.arange(_W, dtype=np.float64)
    angm = 2.0 * np.pi / _W * np.outer(t, t)
    mr = (np.cos(angm) / _W).astype(np.float32)
    mi = (np.sin(angm) / _W).astype(np.float32)
    btop = np.empty((_W, 2 * _W), np.float32)
    bbot = np.empty((_W, 2 * _W), np.float32)
    btop[:, 0::2] = mr
    btop[:, 1::2] = mi
    bbot[:, 0::2] = -mi
    bbot[:, 1::2] = mr
    return dr, di, tr, ti, btop, bbot


def _fft_body(x_ref, dr_ref, di_ref, tr_ref, ti_ref, er_ref, ei_ref):
    a2 = x_ref[0]
    dr = dr_ref[...]
    di = di_ref[...]
    gr = jnp.dot(dr, a2, precision=_PREC, preferred_element_type=jnp.float32)
    gi = jnp.dot(di, a2, precision=_PREC, preferred_element_type=jnp.float32)
    tr = tr_ref[...]
    ti = ti_ref[...]
    hr = gr * tr - gi * ti
    hi = gr * ti + gi * tr
    er_ref[0] = (jnp.dot(hr, dr, precision=_PREC, preferred_element_type=jnp.float32)
                 - jnp.dot(hi, di, precision=_PREC,
                           preferred_element_type=jnp.float32)).astype(jnp.bfloat16)
    ei_ref[0] = (jnp.dot(hr, di, precision=_PREC, preferred_element_type=jnp.float32)
                 + jnp.dot(hi, dr, precision=_PREC,
                           preferred_element_type=jnp.float32)).astype(jnp.bfloat16)


def _idft_body(zre_ref, zim_ref, g_ref, btop_ref, bbot_ref, o_ref):
    gg = g_ref[...].astype(jnp.bfloat16)
    nsig = zre_ref.shape[0]
    pr = (zre_ref[...] * gg[None]).reshape(nsig * gg.shape[0], gg.shape[1])
    pi = (zim_ref[...] * gg[None]).reshape(nsig * gg.shape[0], gg.shape[1])
    acc = jnp.dot(pr, btop_ref[...], preferred_element_type=jnp.float32)
    acc = acc + jnp.dot(pi, bbot_ref[...], preferred_element_type=jnp.float32)
    o_ref[...] = acc.reshape(o_ref.shape)


def _make_sc_gather(f, width):
    bands_per_w = f // _NW
    half = _W // 2
    mesh = plsc.VectorSubcoreMesh(core_axis_name="c", subcore_axis_name="s",
                                  num_cores=_NC, num_subcores=_NS)

    @functools.partial(
        pl.kernel,
        out_type=jax.ShapeDtypeStruct((f, _W, width), jnp.bfloat16),
        mesh=mesh,
        compiler_params=pltpu.CompilerParams(use_tc_tiling_on_sc=False,
                                             needs_layout_passes=False),
        scratch_types=[
            [pltpu.VMEM((16,), jnp.int32) for _ in range(3)],
            [pltpu.VMEM((_W, width), jnp.bfloat16) for _ in range(3)],
            pltpu.SemaphoreType.DMA,
            pltpu.SemaphoreType.DMA,
        ],
    )
    def sc_gather(table_hbm, ix_hbm, out_hbm, tp_vs, win_vs, sem_in, sem_out):
        wid = lax.axis_index("s") * _NC + lax.axis_index("c")
        lane = lax.iota(jnp.int32, 16)
        for j in range(bands_per_w):
            pltpu.sync_copy(ix_hbm.at[wid * bands_per_w + j], tp_vs[j])
        copies = []
        for j in range(bands_per_w):
            tp = jnp.max(jnp.where(lane == 0, tp_vs[j][...], 0))
            copies.append(
                pltpu.async_copy(table_hbm.at[pl.ds(tp, _W)], win_vs[j],
                                 sem_in))
        out_copies = []
        for j in range(bands_per_w):
            band = wid * bands_per_w + j
            copies[j].wait()
            out_copies.append(pltpu.async_copy(
                win_vs[j].at[pl.ds(half, half)],
                out_hbm.at[band, pl.ds(0, half)], sem_out))
            out_copies.append(pltpu.async_copy(
                win_vs[j].at[pl.ds(0, half)],
                out_hbm.at[band, pl.ds(half, half)], sem_out))
        for cp in out_copies:
            cp.wait()

    return sc_gather


def kernel(x, g, win_ix):
    b, c, ls = x.shape
    f, w = g.shape
    bc = b * c
    assert ls == _LS and w == _W

    dr, di, tr, ti, btop, bbot = _dft_consts()

    k2hi = (ls // 2 + _W // 2 + 16 + _N1 - 1) // _N1
    hh = 2 * _N1 + k2hi * _N1

    x3 = x.reshape(bc, _N1, _N1)
    full = pl.BlockSpec((_N1, _N1), lambda i: (0, 0))
    ere, eim = pl.pallas_call(
        _fft_body,
        grid=(bc,),
        in_specs=[pl.BlockSpec((1, _N1, _N1), lambda i: (i, 0, 0)),
                  full, full, full, full],
        out_specs=[pl.BlockSpec((1, _N1, _N1), lambda i: (i, 0, 0))] * 2,
        out_shape=[jax.ShapeDtypeStruct((bc, _N1, _N1), jnp.bfloat16)] * 2,
    )(x3, dr, di, tr, ti)

    low = jnp.stack([ere[:, :, :k2hi], eim[:, :, :k2hi]], axis=-1)
    hal = jnp.stack([ere[:, :, _N1 - 2:], eim[:, :, _N1 - 2:]], axis=-1)
    table = low.transpose(2, 1, 0, 3).reshape(k2hi * _N1, bc * 2)
    halo = hal.transpose(2, 1, 0, 3).reshape(2 * _N1, bc * 2)
    tablep = jnp.concatenate([halo, table], axis=0)

    ix = win_ix[:, :16].astype(jnp.int32)
    z = _make_sc_gather(f, bc * 2)(tablep, ix)
    gt = z.reshape(f, w, bc, 2).transpose(2, 3, 0, 1)
    zre, zim = gt[:, 0], gt[:, 1]

    rows = 4
    fullg = pl.BlockSpec((f, w), lambda i: (0, 0))
    fullm = pl.BlockSpec((w, 2 * w), lambda i: (0, 0))
    out = pl.pallas_call(
        _idft_body,
        grid=(bc // rows,),
        in_specs=[pl.BlockSpec((rows, f, w), lambda i: (i, 0, 0)),
                  pl.BlockSpec((rows, f, w), lambda i: (i, 0, 0)),
                  fullg, fullm, fullm],
        out_specs=pl.BlockSpec((rows, f, 2 * w), lambda i: (i, 0, 0)),
        out_shape=jax.ShapeDtypeStruct((bc, f, 2 * w), jnp.float32),
    )(zre, zim, g,
      jnp.asarray(btop).astype(jnp.bfloat16),
      jnp.asarray(bbot).astype(jnp.bfloat16))

    return out.reshape(b, c, f, w, 2)
